# baseline (speedup 1.0000x reference)
import jax
import jax.numpy as jnp
from jax.experimental import pallas as pl
from jax.experimental.pallas import tpu as pltpu


def _scale8_t_multiw(x_ref, o_hbm, obuf, sem):
    n = obuf.shape[0]
    for i in range(n):
        obuf[i] = jnp.swapaxes(x_ref[i], 0, 1) * 8.0
        pltpu.make_async_copy(obuf.at[i], o_hbm.at[i], sem.at[i]).start()
    for i in range(n):
        pltpu.make_async_copy(obuf.at[i], o_hbm.at[i], sem.at[i]).wait()


def kernel(x):
    B, C, H, W = x.shape
    L = H * W
    xt = jnp.transpose(x, (0, 2, 3, 1)).reshape(B, L, C)
    out = pl.pallas_call(
        _scale8_t_multiw,
        grid=(1,),
        in_specs=[pl.BlockSpec(memory_space=pltpu.MemorySpace.VMEM)],
        out_specs=pl.BlockSpec(memory_space=pltpu.MemorySpace.HBM),
        out_shape=jax.ShapeDtypeStruct((B, C, L), x.dtype),
        scratch_shapes=[
            pltpu.VMEM((B, C, L), jnp.float32),
            pltpu.SemaphoreType.DMA((B,)),
        ],
    )(xt)
    return out
